# Initial kernel scaffold; baseline (speedup 1.0000x reference)
import jax
import jax.numpy as jnp
from jax import lax
from jax.experimental import pallas as pl
from jax.experimental.pallas import tpu as pltpu
from jax.experimental.pallas import tpu_sc as plsc

N = 10000
D = 128
NC = 2
NS = 16
LANES = 128
G = 4

N_PAD = 10240
TRASH = N
ZROWS = N_PAD // NS
WB = N // NS

_mesh = plsc.VectorSubcoreMesh(core_axis_name="c", subcore_axis_name="s")


def _deg_body(dst_hbm, ones_hbm, zeros_hbm, deg_out, idx_v, ones_v, z_v, acc):
    cid = lax.axis_index("c")
    sid = lax.axis_index("s")
    n_rows = dst_hbm.shape[1] // NS
    n_chunks = n_rows // G
    pltpu.sync_copy(ones_hbm, ones_v)
    pltpu.sync_copy(zeros_hbm, z_v)
    pltpu.sync_copy(z_v, acc.at[pl.ds(sid * ZROWS, ZROWS)])
    plsc.subcore_barrier()
    base = sid * n_rows

    def chunk(g, carry):
        pltpu.sync_copy(dst_hbm.at[cid, pl.ds(base + g * G, G)], idx_v)
        for j in range(G):
            pltpu.sync_copy(ones_v, acc.at[idx_v.at[j]], add=True)
        return carry

    lax.fori_loop(0, n_chunks, chunk, 0)
    plsc.subcore_barrier()
    pltpu.sync_copy(acc.at[pl.ds(sid * WB, WB)],
                    deg_out.at[cid, pl.ds(sid * WB, WB)])


def _spmm_body(src_hbm, dst_hbm, uv_hbm, zeros_hbm, agg_out,
               sidx, didx, rows, z_v, acc, sem):
    cid = lax.axis_index("c")
    sid = lax.axis_index("s")
    n_rows = src_hbm.shape[1] // NS
    n_chunks = n_rows // G
    pltpu.sync_copy(zeros_hbm, z_v)
    for t in range(ZROWS // LANES):
        pltpu.sync_copy(z_v, acc.at[pl.ds(sid * ZROWS + t * LANES, LANES)])
    plsc.subcore_barrier()
    base = sid * n_rows

    def chunk(g, carry):
        pltpu.sync_copy(src_hbm.at[cid, pl.ds(base + g * G, G)], sidx)
        pltpu.sync_copy(dst_hbm.at[cid, pl.ds(base + g * G, G)], didx)
        cps = [
            pltpu.async_copy(uv_hbm.at[sidx.at[j]],
                             rows.at[pl.ds(j * LANES, LANES)], sem)
            for j in range(G)
        ]
        for cp in cps:
            cp.wait()
        for j in range(G):
            pltpu.sync_copy(rows.at[pl.ds(j * LANES, LANES)],
                            acc.at[didx.at[j]], add=True)
        return carry

    lax.fori_loop(0, n_chunks, chunk, 0)
    plsc.subcore_barrier()
    pltpu.sync_copy(acc.at[pl.ds(sid * WB, WB)],
                    agg_out.at[cid, pl.ds(sid * WB, WB)])


_deg_call = pl.kernel(
    _deg_body,
    out_type=jax.ShapeDtypeStruct((NC, N, 16), jnp.float32),
    mesh=_mesh,
    scratch_types=[
        pltpu.VMEM((G, LANES), jnp.int32),
        pltpu.VMEM((LANES, 16), jnp.float32),
        pltpu.VMEM((ZROWS, 16), jnp.float32),
        pltpu.VMEM_SHARED((N_PAD, 16), jnp.float32),
    ],
)

_spmm_call = pl.kernel(
    _spmm_body,
    out_type=jax.ShapeDtypeStruct((NC, N, D), jnp.float32),
    mesh=_mesh,
    scratch_types=[
        pltpu.VMEM((G, LANES), jnp.int32),
        pltpu.VMEM((G, LANES), jnp.int32),
        pltpu.VMEM((G * LANES, D), jnp.float32),
        pltpu.VMEM((LANES, D), jnp.float32),
        pltpu.VMEM_SHARED((N_PAD, D), jnp.float32),
        pltpu.SemaphoreType.DMA,
    ],
)


def _gelu(x):
    return x * 0.5 * (1.0 + lax.erf(x * 0.7071067811865476))


def _scale(deg):
    return jnp.where(deg > 0, lax.rsqrt(deg), 0.0)[:, 0:1]


def _pre_body(h_ref, ideg_ref, odeg_ref, wi_ref, bi_ref, wo_ref, bo_ref,
              u_ref, v_ref):
    h = h_ref[...]
    s_src = _scale(odeg_ref[...])
    s_dst = _scale(ideg_ref[...])
    h1 = jnp.dot(h, wi_ref[...], preferred_element_type=jnp.float32) + bi_ref[...]
    h2 = jnp.dot(h, wo_ref[...], preferred_element_type=jnp.float32) + bo_ref[...]
    u_ref[...] = s_src * h1
    v_ref[...] = s_dst * h2


def _post_body(a0_ref, a1_ref, ideg_ref, odeg_ref, wf1_ref, bf1_ref,
               wf2_ref, bf2_ref, o_ref):
    s_dst = _scale(ideg_ref[...])
    s_src = _scale(odeg_ref[...])
    x_in = _gelu(s_dst * a0_ref[...])
    x_out = _gelu(s_src * a1_ref[...])
    cat = jnp.concatenate([x_in, x_out], axis=1)
    z = _gelu(jnp.dot(cat, wf1_ref[...], preferred_element_type=jnp.float32)
              + bf1_ref[...])
    o_ref[...] = (jnp.dot(z, wf2_ref[...], preferred_element_type=jnp.float32)
                  + bf2_ref[...])


BN = 1000


def _row_spec(w):
    return pl.BlockSpec((BN, w), lambda i: (i, 0))


def _full_spec(r, c):
    return pl.BlockSpec((r, c), lambda i: (0, 0))


_pre_call = pl.pallas_call(
    _pre_body,
    grid=(N // BN,),
    in_specs=[
        _row_spec(D), _row_spec(16), _row_spec(16),
        _full_spec(D, D), _full_spec(1, D),
        _full_spec(D, D), _full_spec(1, D),
    ],
    out_specs=[_row_spec(D), _row_spec(D)],
    out_shape=[
        jax.ShapeDtypeStruct((N, D), jnp.float32),
        jax.ShapeDtypeStruct((N, D), jnp.float32),
    ],
)

_post_call = pl.pallas_call(
    _post_body,
    grid=(N // BN,),
    in_specs=[
        _row_spec(D), _row_spec(D), _row_spec(16), _row_spec(16),
        _full_spec(2 * D, D), _full_spec(1, D),
        _full_spec(D, D), _full_spec(1, D),
    ],
    out_specs=_row_spec(D),
    out_shape=jax.ShapeDtypeStruct((N, D), jnp.float32),
)


def kernel(x, edge_index, W_in0, b_in0, W_out0, b_out0, Wf1_0, bf1_0,
           Wf2_0, bf2_0, W_in1, b_in1, W_out1, b_out1, Wf1_1, bf1_1,
           Wf2_1, bf2_1):
    E = edge_index.shape[1]
    blk = NS * LANES * G
    e_pad = -(-E // blk) * blk
    pad = e_pad - E

    row = edge_index[0]
    col = edge_index[1]
    src_p = jnp.concatenate(
        [jnp.stack([row, col + N]),
         jnp.zeros((NC, pad), jnp.int32)], axis=1).reshape(NC, e_pad // LANES, LANES)
    dst_p = jnp.concatenate(
        [jnp.stack([col, row]),
         jnp.full((NC, pad), TRASH, jnp.int32)], axis=1).reshape(NC, e_pad // LANES, LANES)

    ones16 = jnp.ones((LANES, 16), jnp.float32)
    zeros16 = jnp.zeros((ZROWS, 16), jnp.float32)
    zerosD = jnp.zeros((LANES, D), jnp.float32)

    degs = _deg_call(dst_p, ones16, zeros16)
    ideg = degs[0]
    odeg = degs[1]

    params = [
        (W_in0, b_in0, W_out0, b_out0, Wf1_0, bf1_0, Wf2_0, bf2_0),
        (W_in1, b_in1, W_out1, b_out1, Wf1_1, bf1_1, Wf2_1, bf2_1),
    ]
    h = x
    for (Wi, bi, Wo, bo, Wf1, bf1, Wf2, bf2) in params:
        u, v = _pre_call(h, ideg, odeg, Wi.T, bi.reshape(1, D),
                         Wo.T, bo.reshape(1, D))
        uv = jnp.concatenate([u, v], axis=0)
        agg = _spmm_call(src_p, dst_p, uv, zerosD)
        h = _post_call(agg[0], agg[1], ideg, odeg, Wf1.T, bf1.reshape(1, D),
                       Wf2.T, bf2.reshape(1, D))
    return h


# speedup vs baseline: 4.5416x; 4.5416x over previous
import functools

import jax
import jax.numpy as jnp
from jax import lax
from jax.experimental import pallas as pl
from jax.experimental.pallas import tpu as pltpu
from jax.experimental.pallas import tpu_sc as plsc

N = 10000
D = 128
NC = 2
NS = 16
LANES = 128
G = 4

N_PAD = 10240
TRASH = N
ZROWS = N_PAD // NS

def _deg_body(dst_hbm, ones_hbm, zeros_hbm, deg_out, idx_v, ones_v, z_v, acc):
    cid = lax.axis_index("c")
    sid = lax.axis_index("s")
    n_rows = dst_hbm.shape[1] // NS
    pltpu.sync_copy(ones_hbm, ones_v)
    pltpu.sync_copy(zeros_hbm, z_v)
    pltpu.sync_copy(z_v, acc.at[pl.ds(sid * ZROWS, ZROWS)])
    plsc.subcore_barrier()
    base = sid * n_rows

    def chunk(g, carry):
        pltpu.sync_copy(dst_hbm.at[cid, base + g], idx_v)
        pltpu.sync_copy(ones_v, acc.at[idx_v], add=True)
        return carry

    lax.fori_loop(0, n_rows, chunk, 0)
    plsc.subcore_barrier()
    pltpu.sync_copy(acc.at[pl.ds(sid * ZROWS, ZROWS)],
                    deg_out.at[cid, pl.ds(sid * ZROWS, ZROWS)])


HD = D // 2


def _spmm_body(src_hbm, dst_hbm, uv0_hbm, uv1_hbm, zeros_hbm, agg_out,
               sidx, didx, rows, z_v, acc, sem):
    cid = lax.axis_index("c")
    sid = lax.axis_index("s")
    n_rows = src_hbm.shape[1] // NS
    base = sid * n_rows
    pltpu.sync_copy(zeros_hbm, z_v)

    for p, uv_hbm in enumerate((uv0_hbm, uv1_hbm)):
        for t in range(ZROWS // LANES):
            pltpu.sync_copy(z_v, acc.at[pl.ds(sid * ZROWS + t * LANES, LANES)])
        plsc.subcore_barrier()

        def chunk(g, carry):
            pltpu.sync_copy(src_hbm.at[cid, base + g], sidx)
            pltpu.sync_copy(dst_hbm.at[cid, base + g], didx)
            pltpu.async_copy(uv_hbm.at[sidx], rows, sem).wait()
            pltpu.sync_copy(rows, acc.at[didx], add=True)
            return carry

        lax.fori_loop(0, n_rows, chunk, 0)
        plsc.subcore_barrier()
        pltpu.sync_copy(acc.at[pl.ds(sid * ZROWS, ZROWS)],
                        agg_out.at[p, cid, pl.ds(sid * ZROWS, ZROWS)])


@functools.cache
def _sc_calls():
    mesh = plsc.VectorSubcoreMesh(core_axis_name="c", subcore_axis_name="s")
    deg_call = pl.kernel(
        _deg_body,
        out_type=jax.ShapeDtypeStruct((NC, N_PAD, 16), jnp.float32),
        mesh=mesh,
        scratch_types=[
            pltpu.VMEM((LANES,), jnp.int32),
            pltpu.VMEM((LANES, 16), jnp.float32),
            pltpu.VMEM((ZROWS, 16), jnp.float32),
            pltpu.VMEM_SHARED((N_PAD, 16), jnp.float32),
        ],
        compiler_params=pltpu.CompilerParams(use_tc_tiling_on_sc=False),
    )
    spmm_call = pl.kernel(
        _spmm_body,
        out_type=jax.ShapeDtypeStruct((2, NC, N_PAD, HD), jnp.float32),
        mesh=mesh,
        scratch_types=[
            pltpu.VMEM((LANES,), jnp.int32),
            pltpu.VMEM((LANES,), jnp.int32),
            pltpu.VMEM((LANES, HD), jnp.float32),
            pltpu.VMEM((LANES, HD), jnp.float32),
            pltpu.VMEM_SHARED((N_PAD, HD), jnp.float32),
            pltpu.SemaphoreType.DMA,
        ],
        compiler_params=pltpu.CompilerParams(use_tc_tiling_on_sc=False),
    )
    return deg_call, spmm_call


def _gelu(x):
    return x * 0.5 * (1.0 + lax.erf(x * 0.7071067811865476))


def _scale(deg):
    return jnp.where(deg > 0, lax.rsqrt(deg), 0.0)[:, 0:1]


def _pre_body(h_ref, ideg_ref, odeg_ref, wi_ref, bi_ref, wo_ref, bo_ref,
              u_ref, v_ref):
    h = h_ref[...]
    s_src = _scale(odeg_ref[...])
    s_dst = _scale(ideg_ref[...])
    h1 = jnp.dot(h, wi_ref[...], preferred_element_type=jnp.float32) + bi_ref[...]
    h2 = jnp.dot(h, wo_ref[...], preferred_element_type=jnp.float32) + bo_ref[...]
    u_ref[...] = s_src * h1
    v_ref[...] = s_dst * h2


def _post_body(a0_ref, a1_ref, ideg_ref, odeg_ref, wf1_ref, bf1_ref,
               wf2_ref, bf2_ref, o_ref):
    s_dst = _scale(ideg_ref[...])
    s_src = _scale(odeg_ref[...])
    x_in = _gelu(s_dst * a0_ref[...])
    x_out = _gelu(s_src * a1_ref[...])
    cat = jnp.concatenate([x_in, x_out], axis=1)
    z = _gelu(jnp.dot(cat, wf1_ref[...], preferred_element_type=jnp.float32)
              + bf1_ref[...])
    o_ref[...] = (jnp.dot(z, wf2_ref[...], preferred_element_type=jnp.float32)
                  + bf2_ref[...])


BN = 1000


def _row_spec(w):
    return pl.BlockSpec((BN, w), lambda i: (i, 0))


def _full_spec(r, c):
    return pl.BlockSpec((r, c), lambda i: (0, 0))


_pre_call = pl.pallas_call(
    _pre_body,
    grid=(N // BN,),
    in_specs=[
        _row_spec(D), _row_spec(16), _row_spec(16),
        _full_spec(D, D), _full_spec(1, D),
        _full_spec(D, D), _full_spec(1, D),
    ],
    out_specs=[_row_spec(D), _row_spec(D)],
    out_shape=[
        jax.ShapeDtypeStruct((N, D), jnp.float32),
        jax.ShapeDtypeStruct((N, D), jnp.float32),
    ],
)

_post_call = pl.pallas_call(
    _post_body,
    grid=(N // BN,),
    in_specs=[
        _row_spec(D), _row_spec(D), _row_spec(16), _row_spec(16),
        _full_spec(2 * D, D), _full_spec(1, D),
        _full_spec(D, D), _full_spec(1, D),
    ],
    out_specs=_row_spec(D),
    out_shape=jax.ShapeDtypeStruct((N, D), jnp.float32),
)


def kernel(x, edge_index, W_in0, b_in0, W_out0, b_out0, Wf1_0, bf1_0,
           Wf2_0, bf2_0, W_in1, b_in1, W_out1, b_out1, Wf1_1, bf1_1,
           Wf2_1, bf2_1):
    E = edge_index.shape[1]
    blk = NS * LANES * G
    e_pad = -(-E // blk) * blk
    pad = e_pad - E

    row = edge_index[0]
    col = edge_index[1]
    src_p = jnp.concatenate(
        [jnp.stack([row, col + N]),
         jnp.zeros((NC, pad), jnp.int32)], axis=1).reshape(NC, e_pad // LANES, LANES)
    dst_p = jnp.concatenate(
        [jnp.stack([col, row]),
         jnp.full((NC, pad), TRASH, jnp.int32)], axis=1).reshape(NC, e_pad // LANES, LANES)

    ones16 = jnp.ones((LANES, 16), jnp.float32)
    zeros16 = jnp.zeros((ZROWS, 16), jnp.float32)
    zerosD = jnp.zeros((LANES, HD), jnp.float32)

    deg_call, spmm_call = _sc_calls()
    degs = deg_call(dst_p, ones16, zeros16)
    ideg = degs[0, :N]
    odeg = degs[1, :N]

    params = [
        (W_in0, b_in0, W_out0, b_out0, Wf1_0, bf1_0, Wf2_0, bf2_0),
        (W_in1, b_in1, W_out1, b_out1, Wf1_1, bf1_1, Wf2_1, bf2_1),
    ]
    h = x
    for (Wi, bi, Wo, bo, Wf1, bf1, Wf2, bf2) in params:
        u, v = _pre_call(h, ideg, odeg, Wi.T, bi.reshape(1, D),
                         Wo.T, bo.reshape(1, D))
        uv = jnp.concatenate([u, v], axis=0)
        agg = spmm_call(src_p, dst_p, uv[:, :HD], uv[:, HD:], zerosD)
        a0 = jnp.concatenate([agg[0, 0, :N], agg[1, 0, :N]], axis=1)
        a1 = jnp.concatenate([agg[0, 1, :N], agg[1, 1, :N]], axis=1)
        h = _post_call(a0, a1, ideg, odeg, Wf1.T, bf1.reshape(1, D),
                       Wf2.T, bf2.reshape(1, D))
    return h


# speedup vs baseline: 5.9123x; 1.3018x over previous
import functools

import jax
import jax.numpy as jnp
from jax import lax
from jax.experimental import pallas as pl
from jax.experimental.pallas import tpu as pltpu
from jax.experimental.pallas import tpu_sc as plsc

N = 10000
D = 128
NC = 2
NS = 16
LANES = 128
G = 4

N_PAD = 10240
TRASH = N
ZROWS = N_PAD // NS

def _deg_body(dst_hbm, ones_hbm, zeros_hbm, deg_out, idx_v, ones_v, z_v, acc):
    cid = lax.axis_index("c")
    sid = lax.axis_index("s")
    n_rows = dst_hbm.shape[1] // NS
    pltpu.sync_copy(ones_hbm, ones_v)
    pltpu.sync_copy(zeros_hbm, z_v)
    pltpu.sync_copy(z_v, acc.at[pl.ds(sid * ZROWS, ZROWS)])
    plsc.subcore_barrier()
    base = sid * n_rows

    def chunk(g, carry):
        pltpu.sync_copy(dst_hbm.at[cid, base + g], idx_v)
        pltpu.sync_copy(ones_v, acc.at[idx_v], add=True)
        return carry

    lax.fori_loop(0, n_rows, chunk, 0)
    plsc.subcore_barrier()
    pltpu.sync_copy(acc.at[pl.ds(sid * ZROWS, ZROWS)],
                    deg_out.at[cid, pl.ds(sid * ZROWS, ZROWS)])


HD = D // 2
NB = 8


def _spmm_body(src_hbm, dst_hbm, uv0_hbm, uv1_hbm, zeros_hbm, agg_out,
               sidx_l, didx_l, rows_l, z_v, acc, isem, gsem_l):
    cid = lax.axis_index("c")
    sid = lax.axis_index("s")
    n_rows = src_hbm.shape[1] // NS
    base = sid * n_rows
    pltpu.sync_copy(zeros_hbm, z_v)

    for p, uv_hbm in enumerate((uv0_hbm, uv1_hbm)):
        for t in range(ZROWS // LANES):
            pltpu.sync_copy(z_v, acc.at[pl.ds(sid * ZROWS + t * LANES, LANES)])
        plsc.subcore_barrier()

        def chunk(s, carry):
            g0 = base + s * NB
            icps = []
            for b in range(NB):
                icps.append(pltpu.async_copy(src_hbm.at[cid, g0 + b],
                                             sidx_l[b], isem))
                icps.append(pltpu.async_copy(dst_hbm.at[cid, g0 + b],
                                             didx_l[b], isem))
            for cp in icps:
                cp.wait()
            gcps = [pltpu.async_copy(uv_hbm.at[sidx_l[b]], rows_l[b],
                                     gsem_l[b]) for b in range(NB)]
            for b in range(NB):
                gcps[b].wait()
                pltpu.sync_copy(rows_l[b], acc.at[didx_l[b]], add=True)
            return carry

        lax.fori_loop(0, n_rows // NB, chunk, 0)
        plsc.subcore_barrier()
        pltpu.sync_copy(acc.at[pl.ds(sid * ZROWS, ZROWS)],
                        agg_out.at[p, cid, pl.ds(sid * ZROWS, ZROWS)])


@functools.cache
def _sc_calls():
    mesh = plsc.VectorSubcoreMesh(core_axis_name="c", subcore_axis_name="s")
    deg_call = pl.kernel(
        _deg_body,
        out_type=jax.ShapeDtypeStruct((NC, N_PAD, 16), jnp.float32),
        mesh=mesh,
        scratch_types=[
            pltpu.VMEM((LANES,), jnp.int32),
            pltpu.VMEM((LANES, 16), jnp.float32),
            pltpu.VMEM((ZROWS, 16), jnp.float32),
            pltpu.VMEM_SHARED((N_PAD, 16), jnp.float32),
        ],
        compiler_params=pltpu.CompilerParams(use_tc_tiling_on_sc=False),
    )
    spmm_call = pl.kernel(
        _spmm_body,
        out_type=jax.ShapeDtypeStruct((2, NC, N_PAD, HD), jnp.float32),
        mesh=mesh,
        scratch_types=[
            [pltpu.VMEM((LANES,), jnp.int32) for _ in range(NB)],
            [pltpu.VMEM((LANES,), jnp.int32) for _ in range(NB)],
            [pltpu.VMEM((LANES, HD), jnp.float32) for _ in range(NB)],
            pltpu.VMEM((LANES, HD), jnp.float32),
            pltpu.VMEM_SHARED((N_PAD, HD), jnp.float32),
            pltpu.SemaphoreType.DMA,
            [pltpu.SemaphoreType.DMA for _ in range(NB)],
        ],
        compiler_params=pltpu.CompilerParams(use_tc_tiling_on_sc=False),
    )
    return deg_call, spmm_call


def _gelu(x):
    return x * 0.5 * (1.0 + lax.erf(x * 0.7071067811865476))


def _scale(deg):
    return jnp.where(deg > 0, lax.rsqrt(deg), 0.0)[:, 0:1]


def _pre_body(h_ref, ideg_ref, odeg_ref, wi_ref, bi_ref, wo_ref, bo_ref,
              u_ref, v_ref):
    h = h_ref[...]
    s_src = _scale(odeg_ref[...])
    s_dst = _scale(ideg_ref[...])
    h1 = jnp.dot(h, wi_ref[...], preferred_element_type=jnp.float32) + bi_ref[...]
    h2 = jnp.dot(h, wo_ref[...], preferred_element_type=jnp.float32) + bo_ref[...]
    u_ref[...] = s_src * h1
    v_ref[...] = s_dst * h2


def _post_body(a0_ref, a1_ref, ideg_ref, odeg_ref, wf1_ref, bf1_ref,
               wf2_ref, bf2_ref, o_ref):
    s_dst = _scale(ideg_ref[...])
    s_src = _scale(odeg_ref[...])
    x_in = _gelu(s_dst * a0_ref[...])
    x_out = _gelu(s_src * a1_ref[...])
    cat = jnp.concatenate([x_in, x_out], axis=1)
    z = _gelu(jnp.dot(cat, wf1_ref[...], preferred_element_type=jnp.float32)
              + bf1_ref[...])
    o_ref[...] = (jnp.dot(z, wf2_ref[...], preferred_element_type=jnp.float32)
                  + bf2_ref[...])


BN = 1000


def _row_spec(w):
    return pl.BlockSpec((BN, w), lambda i: (i, 0))


def _full_spec(r, c):
    return pl.BlockSpec((r, c), lambda i: (0, 0))


_pre_call = pl.pallas_call(
    _pre_body,
    grid=(N // BN,),
    in_specs=[
        _row_spec(D), _row_spec(16), _row_spec(16),
        _full_spec(D, D), _full_spec(1, D),
        _full_spec(D, D), _full_spec(1, D),
    ],
    out_specs=[_row_spec(D), _row_spec(D)],
    out_shape=[
        jax.ShapeDtypeStruct((N, D), jnp.float32),
        jax.ShapeDtypeStruct((N, D), jnp.float32),
    ],
)

_post_call = pl.pallas_call(
    _post_body,
    grid=(N // BN,),
    in_specs=[
        _row_spec(D), _row_spec(D), _row_spec(16), _row_spec(16),
        _full_spec(2 * D, D), _full_spec(1, D),
        _full_spec(D, D), _full_spec(1, D),
    ],
    out_specs=_row_spec(D),
    out_shape=jax.ShapeDtypeStruct((N, D), jnp.float32),
)


def kernel(x, edge_index, W_in0, b_in0, W_out0, b_out0, Wf1_0, bf1_0,
           Wf2_0, bf2_0, W_in1, b_in1, W_out1, b_out1, Wf1_1, bf1_1,
           Wf2_1, bf2_1):
    E = edge_index.shape[1]
    blk = NS * LANES * G
    e_pad = -(-E // blk) * blk
    pad = e_pad - E

    row = edge_index[0]
    col = edge_index[1]
    src_p = jnp.concatenate(
        [jnp.stack([row, col + N]),
         jnp.zeros((NC, pad), jnp.int32)], axis=1).reshape(NC, e_pad // LANES, LANES)
    dst_p = jnp.concatenate(
        [jnp.stack([col, row]),
         jnp.full((NC, pad), TRASH, jnp.int32)], axis=1).reshape(NC, e_pad // LANES, LANES)

    ones16 = jnp.ones((LANES, 16), jnp.float32)
    zeros16 = jnp.zeros((ZROWS, 16), jnp.float32)
    zerosD = jnp.zeros((LANES, HD), jnp.float32)

    deg_call, spmm_call = _sc_calls()
    degs = deg_call(dst_p, ones16, zeros16)
    ideg = degs[0, :N]
    odeg = degs[1, :N]

    params = [
        (W_in0, b_in0, W_out0, b_out0, Wf1_0, bf1_0, Wf2_0, bf2_0),
        (W_in1, b_in1, W_out1, b_out1, Wf1_1, bf1_1, Wf2_1, bf2_1),
    ]
    h = x
    for (Wi, bi, Wo, bo, Wf1, bf1, Wf2, bf2) in params:
        u, v = _pre_call(h, ideg, odeg, Wi.T, bi.reshape(1, D),
                         Wo.T, bo.reshape(1, D))
        uv = jnp.concatenate([u, v], axis=0)
        agg = spmm_call(src_p, dst_p, uv[:, :HD], uv[:, HD:], zerosD)
        a0 = jnp.concatenate([agg[0, 0, :N], agg[1, 0, :N]], axis=1)
        a1 = jnp.concatenate([agg[0, 1, :N], agg[1, 1, :N]], axis=1)
        h = _post_call(a0, a1, ideg, odeg, Wf1.T, bf1.reshape(1, D),
                       Wf2.T, bf2.reshape(1, D))
    return h
